# baseline (device time: 17116 ns/iter reference)
import jax
import jax.numpy as jnp
from jax import lax
from jax.experimental import pallas as pl
from jax.experimental.pallas import tpu as pltpu

N_DEV = 8
GRID = 8


def kernel(x):
    m_per, n = x.shape
    chunk = m_per // GRID

    def body(
        x_ref, out_ref, run_val, run_idx, run_idx2, gather_ref, vmem_blocker,
        send_sems, recv_sems
    ):
        g = pl.program_id(0)
        my_pos = lax.axis_index("i")
        barrier_sem = pltpu.get_barrier_semaphore()

        @pl.when(g == 0)
        def _():
            vmem_blocker[0:1, 0:128] = jnp.zeros((1, 128), jnp.float32)

        @pl.when(g == 0)
        def _():
            for p in range(N_DEV):
                @pl.when(my_pos != p)
                def _():
                    pl.semaphore_signal(
                        barrier_sem,
                        inc=1,
                        device_id=(p,),
                        device_id_type=pl.DeviceIdType.MESH,
                    )

        xv = x_ref[:, :]
        bval = jnp.max(xv, axis=0)
        rows = lax.broadcasted_iota(jnp.int32, (chunk, n), 0)
        bidx = (
            jnp.min(jnp.where(xv == bval[None, :], rows, chunk), axis=0)
            + g * chunk
        )

        @pl.when(g == 0)
        def _():
            run_val[0, :] = bval
            run_idx[0, :] = bidx

        @pl.when(g > 0)
        def _():
            rv = run_val[0, :]
            better = bval > rv
            run_val[0, :] = jnp.where(better, bval, rv)
            run_idx[0, :] = jnp.where(better, bidx, run_idx[0, :])

        @pl.when(g == GRID - 1)
        def _():
            gidx = (my_pos * m_per + run_idx[0, :]).astype(jnp.float32)
            partial = jnp.stack([run_val[0, :], gidx], axis=0)
            gather_ref[pl.ds(my_pos, 1)] = partial[None]

            pl.semaphore_wait(barrier_sem, N_DEV - 1)

            for p in range(N_DEV):
                @pl.when(my_pos != p)
                def _():
                    rdma = pltpu.make_async_remote_copy(
                        src_ref=gather_ref.at[my_pos],
                        dst_ref=gather_ref.at[my_pos],
                        send_sem=send_sems.at[p],
                        recv_sem=recv_sems.at[my_pos],
                        device_id=(p,),
                        device_id_type=pl.DeviceIdType.MESH,
                    )
                    rdma.start()

            for p in range(N_DEV):
                @pl.when(my_pos != p)
                def _():
                    recv_done = pltpu.make_async_remote_copy(
                        src_ref=gather_ref.at[p],
                        dst_ref=gather_ref.at[p],
                        send_sem=send_sems.at[p],
                        recv_sem=recv_sems.at[p],
                        device_id=(p,),
                        device_id_type=pl.DeviceIdType.MESH,
                    )
                    recv_done.wait_recv()

                pv = gather_ref[p, 0, :]
                pi = gather_ref[p, 1, :]
                if p == 0:
                    run_val[0, :] = pv
                    run_idx2[0, :] = pi
                else:
                    rv = run_val[0, :]
                    better = pv > rv
                    run_val[0, :] = jnp.where(better, pv, rv)
                    run_idx2[0, :] = jnp.where(better, pi, run_idx2[0, :])

            out_ref[:, :] = jnp.stack([run_val[0, :], run_idx2[0, :]], axis=0)

            for p in range(N_DEV):
                @pl.when(my_pos != p)
                def _():
                    send_done = pltpu.make_async_remote_copy(
                        src_ref=gather_ref.at[my_pos],
                        dst_ref=gather_ref.at[my_pos],
                        send_sem=send_sems.at[p],
                        recv_sem=recv_sems.at[my_pos],
                        device_id=(p,),
                        device_id_type=pl.DeviceIdType.MESH,
                    )
                    send_done.wait_send()

    return pl.pallas_call(
        body,
        grid=(GRID,),
        out_shape=jax.ShapeDtypeStruct((2, n), jnp.float32),
        in_specs=[pl.BlockSpec((chunk, n), lambda g: (g, 0))],
        out_specs=pl.BlockSpec((2, n), lambda g: (0, 0)),
        scratch_shapes=[
            pltpu.VMEM((1, n), jnp.float32),
            pltpu.VMEM((1, n), jnp.int32),
            pltpu.VMEM((1, n), jnp.float32),
            pltpu.VMEM((N_DEV, 2, n), jnp.float32),
            pltpu.VMEM((44 * 2048, 128), jnp.float32),
            pltpu.SemaphoreType.DMA((N_DEV,)),
            pltpu.SemaphoreType.DMA((N_DEV,)),
        ],
        compiler_params=pltpu.CompilerParams(
            collective_id=0,
            vmem_limit_bytes=58 * 1024 * 1024,
        ),
    )(x)


# device time: 15313 ns/iter; 1.1177x vs baseline; 1.1177x over previous
import jax
import jax.numpy as jnp
from jax import lax
from jax.experimental import pallas as pl
from jax.experimental.pallas import tpu as pltpu

N_DEV = 8
GRID = 2


def kernel(x):
    m_per, n = x.shape
    chunk = m_per // GRID

    def body(
        x_ref, out_ref, run_val, run_idx, run_idx2, gather_ref,
        vmem_blocker, send_sems, recv_sems
    ):
        g = pl.program_id(0)
        my_pos = lax.axis_index("i")
        barrier_sem = pltpu.get_barrier_semaphore()

        @pl.when(g == 0)
        def _():
            vmem_blocker[0:1, 0:128] = jnp.zeros((1, 128), jnp.float32)

        @pl.when(g == 0)
        def _():
            for p in range(N_DEV):
                @pl.when(my_pos != p)
                def _():
                    pl.semaphore_signal(
                        barrier_sem,
                        inc=1,
                        device_id=(p,),
                        device_id_type=pl.DeviceIdType.MESH,
                    )

        xv = x_ref[:, :]
        bval = jnp.max(xv, axis=0)
        rows = lax.broadcasted_iota(jnp.int32, (chunk, n), 0)
        bidx = (
            jnp.min(jnp.where(xv == bval[None, :], rows, chunk), axis=0)
            + g * chunk
        )

        @pl.when(g == 0)
        def _():
            run_val[0, :] = bval
            run_idx[0, :] = bidx

        @pl.when(g > 0)
        def _():
            rv = run_val[0, :]
            better = bval > rv
            run_val[0, :] = jnp.where(better, bval, rv)
            run_idx[0, :] = jnp.where(better, bidx, run_idx[0, :])

        @pl.when(g == GRID - 1)
        def _():
            gidx = (my_pos * m_per + run_idx[0, :]).astype(jnp.float32)
            partial = jnp.stack([run_val[0, :], gidx], axis=0)
            gather_ref[pl.ds(my_pos, 1)] = partial[None]

            pl.semaphore_wait(barrier_sem, N_DEV - 1)

            for p in range(N_DEV):
                @pl.when(my_pos != p)
                def _():
                    rdma = pltpu.make_async_remote_copy(
                        src_ref=gather_ref.at[my_pos],
                        dst_ref=gather_ref.at[my_pos],
                        send_sem=send_sems.at[p],
                        recv_sem=recv_sems.at[my_pos],
                        device_id=(p,),
                        device_id_type=pl.DeviceIdType.MESH,
                    )
                    rdma.start()

            for p in range(N_DEV):
                @pl.when(my_pos != p)
                def _():
                    recv_done = pltpu.make_async_remote_copy(
                        src_ref=gather_ref.at[p],
                        dst_ref=gather_ref.at[p],
                        send_sem=send_sems.at[p],
                        recv_sem=recv_sems.at[p],
                        device_id=(p,),
                        device_id_type=pl.DeviceIdType.MESH,
                    )
                    recv_done.wait_recv()

                pv = gather_ref[p, 0, :]
                pi = gather_ref[p, 1, :]
                if p == 0:
                    run_val[0, :] = pv
                    run_idx2[0, :] = pi
                else:
                    rv = run_val[0, :]
                    better = pv > rv
                    run_val[0, :] = jnp.where(better, pv, rv)
                    run_idx2[0, :] = jnp.where(better, pi, run_idx2[0, :])

            out_ref[:, :] = jnp.stack([run_val[0, :], run_idx2[0, :]], axis=0)

            for p in range(N_DEV):
                @pl.when(my_pos != p)
                def _():
                    send_done = pltpu.make_async_remote_copy(
                        src_ref=gather_ref.at[my_pos],
                        dst_ref=gather_ref.at[my_pos],
                        send_sem=send_sems.at[p],
                        recv_sem=recv_sems.at[my_pos],
                        device_id=(p,),
                        device_id_type=pl.DeviceIdType.MESH,
                    )
                    send_done.wait_send()

    return pl.pallas_call(
        body,
        grid=(GRID,),
        out_shape=jax.ShapeDtypeStruct((2, n), jnp.float32),
        in_specs=[pl.BlockSpec((chunk, n), lambda g: (g, 0))],
        out_specs=pl.BlockSpec((2, n), lambda g: (0, 0)),
        scratch_shapes=[
            pltpu.VMEM((1, n), jnp.float32),
            pltpu.VMEM((1, n), jnp.int32),
            pltpu.VMEM((1, n), jnp.float32),
            pltpu.VMEM((N_DEV, 2, n), jnp.float32),
            pltpu.VMEM((34 * 2048, 128), jnp.float32),
            pltpu.SemaphoreType.DMA((N_DEV,)),
            pltpu.SemaphoreType.DMA((N_DEV,)),
        ],
        compiler_params=pltpu.CompilerParams(
            collective_id=0,
            vmem_limit_bytes=58 * 1024 * 1024,
        ),
    )(x)
